# Initial kernel scaffold; baseline (speedup 1.0000x reference)
import jax
import jax.numpy as jnp
from jax.experimental import pallas as pl


def kernel(node_input, edge_src, edge_dst, edge_attr, edge_scalar_attr, W_node, W1, W2, Wp, W_out):
    raise NotImplementedError("write your pallas kernel here")


# speedup vs baseline: 1.1550x; 1.1550x over previous
import functools

import jax
import jax.numpy as jnp
import numpy as np
from jax import lax
from jax.experimental import pallas as pl
from jax.experimental.pallas import tpu as pltpu
from jax.experimental.pallas import tpu_sc as plsc

N_NODES = 10000
N_EDGES = 320000
D_FEAT = 128
D_EDGE = 16
D_SC = 8
D_HID = 64
D_OUT = 128
MIXING = 0.15
NUM_NEIGHBORS = 32.0

NC = 2
NS = 16
NW = NC * NS
L = 16

EDGES_PER_TILE = N_EDGES // NW
CHUNK = 80
NCHUNKS = EDGES_PER_TILE // CHUNK
ROWS_PER_TILE = 624
TAIL_ROWS = N_NODES - NS * ROWS_PER_TILE


def _node_linear_body(x_ref, w_ref, feat_ref, self_ref):
    y = jnp.dot(x_ref[...], w_ref[...], preferred_element_type=jnp.float32,
                precision=lax.Precision.HIGHEST)
    feat_ref[...] = y[:, :D_FEAT]
    self_ref[...] = y[:, D_FEAT:]


def _node_linear(node_input, W_node):
    blk = 1000
    grid = N_NODES // blk
    return pl.pallas_call(
        _node_linear_body,
        grid=(grid,),
        in_specs=[
            pl.BlockSpec((blk, D_FEAT), lambda i: (i, 0)),
            pl.BlockSpec((D_FEAT, D_FEAT + D_OUT), lambda i: (0, 0)),
        ],
        out_specs=[
            pl.BlockSpec((blk, D_FEAT), lambda i: (i, 0)),
            pl.BlockSpec((blk, D_OUT), lambda i: (i, 0)),
        ],
        out_shape=[
            jax.ShapeDtypeStruct((N_NODES, D_FEAT), jnp.float32),
            jax.ShapeDtypeStruct((N_NODES, D_OUT), jnp.float32),
        ],
    )(node_input, W_node)


def _radial_body(sc_ref, ea_ref, w1_ref, w2_ref, wp_ref, out_ref):
    h = jax.nn.gelu(jnp.dot(sc_ref[...], w1_ref[...],
                            preferred_element_type=jnp.float32,
                            precision=lax.Precision.HIGHEST))
    h = jax.nn.gelu(jnp.dot(h, w2_ref[...],
                            preferred_element_type=jnp.float32,
                            precision=lax.Precision.HIGHEST))
    w = jnp.dot(h, wp_ref[...], preferred_element_type=jnp.float32,
                precision=lax.Precision.HIGHEST) * (1.0 / np.sqrt(D_HID))
    out_ref[...] = w * ea_ref[...]


def _radial(edge_scalar_attr, edge_attr, W1, W2, Wp):
    blk = 8000
    grid = N_EDGES // blk
    return pl.pallas_call(
        _radial_body,
        grid=(grid,),
        in_specs=[
            pl.BlockSpec((blk, D_SC), lambda i: (i, 0)),
            pl.BlockSpec((blk, D_EDGE), lambda i: (i, 0)),
            pl.BlockSpec((D_SC, D_HID), lambda i: (0, 0)),
            pl.BlockSpec((D_HID, D_HID), lambda i: (0, 0)),
            pl.BlockSpec((D_HID, D_EDGE), lambda i: (0, 0)),
        ],
        out_specs=pl.BlockSpec((blk, D_EDGE), lambda i: (i, 0)),
        out_shape=jax.ShapeDtypeStruct((N_EDGES, D_EDGE), jnp.float32),
    )(edge_scalar_attr, edge_attr, W1, W2, Wp)


def _sc_body(feat_hbm, src_hbm, dst_hbm, coeff_hbm, zeros_hbm, out_hbm,
             idx_v, dst_v, coeff_v, rows_v, msg_v, acc_shared):
    c = lax.axis_index("c")
    s = lax.axis_index("s")
    wid = c * NS + s

    pltpu.sync_copy(zeros_hbm.at[pl.ds(s * ROWS_PER_TILE, ROWS_PER_TILE)],
                    acc_shared.at[pl.ds(s * ROWS_PER_TILE, ROWS_PER_TILE)])

    @pl.when(s == NS - 1)
    def _():
        pltpu.sync_copy(
            zeros_hbm.at[pl.ds(NS * ROWS_PER_TILE, TAIL_ROWS)],
            acc_shared.at[pl.ds(NS * ROWS_PER_TILE, TAIL_ROWS)])

    plsc.subcore_barrier()

    tile_base = wid * EDGES_PER_TILE

    def chunk_body(ci, carry):
        base = tile_base + ci * CHUNK
        pltpu.sync_copy(src_hbm.at[pl.ds(base, CHUNK)], idx_v)
        pltpu.sync_copy(dst_hbm.at[pl.ds(base, CHUNK)], dst_v)
        pltpu.sync_copy(coeff_hbm.at[pl.ds(base * D_EDGE, CHUNK * D_EDGE)],
                        coeff_v)
        pltpu.sync_copy(feat_hbm.at[idx_v], rows_v)

        lane = lax.iota(jnp.int32, L)
        half = lax.shift_right_logical(lane, 3)

        def edge_body(e, carry2):
            cbase = e * D_EDGE
            for v in range(D_FEAT // L):
                cidx = cbase + 2 * v + half
                cexp = plsc.load_gather(coeff_v, [cidx])
                row = rows_v[e, pl.ds(v * L, L)]
                msg_v[e, pl.ds(v * L, L)] = row * cexp
            return carry2

        lax.fori_loop(0, CHUNK, edge_body, 0, unroll=False)

        pltpu.sync_copy(msg_v, acc_shared.at[dst_v], add=True)
        return carry

    lax.fori_loop(0, NCHUNKS, chunk_body, 0, unroll=False)

    plsc.subcore_barrier()
    pltpu.sync_copy(acc_shared.at[pl.ds(s * ROWS_PER_TILE, ROWS_PER_TILE)],
                    out_hbm.at[c, pl.ds(s * ROWS_PER_TILE, ROWS_PER_TILE)])

    @pl.when(s == NS - 1)
    def _():
        pltpu.sync_copy(
            acc_shared.at[pl.ds(NS * ROWS_PER_TILE, TAIL_ROWS)],
            out_hbm.at[c, pl.ds(NS * ROWS_PER_TILE, TAIL_ROWS)])


def _sc_scatter(node_features, edge_src, edge_dst, coeff_flat, zeros):
    mesh = plsc.VectorSubcoreMesh(core_axis_name="c", subcore_axis_name="s",
                                  num_cores=NC, num_subcores=NS)
    kern = pl.kernel(
        _sc_body,
        out_type=jax.ShapeDtypeStruct((NC, N_NODES, D_FEAT), jnp.float32),
        mesh=mesh,
        compiler_params=pltpu.CompilerParams(needs_layout_passes=False),
        scratch_types=[
            pltpu.VMEM((CHUNK,), jnp.int32),
            pltpu.VMEM((CHUNK,), jnp.int32),
            pltpu.VMEM((CHUNK * D_EDGE,), jnp.float32),
            pltpu.VMEM((CHUNK, D_FEAT), jnp.float32),
            pltpu.VMEM((CHUNK, D_FEAT), jnp.float32),
            pltpu.VMEM_SHARED((N_NODES, D_FEAT), jnp.float32),
        ],
    )
    return kern(node_features, edge_src, edge_dst, coeff_flat, zeros)


def _out_body(agg_ref, self_ref, wout_ref, out_ref):
    c_self = np.float32(np.sqrt(1.0 - MIXING))
    c_conv = np.float32(np.sqrt(MIXING) / np.sqrt(NUM_NEIGHBORS))
    agg = agg_ref[0] + agg_ref[1]
    conv = jnp.dot(agg, wout_ref[...], preferred_element_type=jnp.float32,
                   precision=lax.Precision.HIGHEST)
    out_ref[...] = c_self * self_ref[...] + c_conv * conv


def _out_linear(agg_pair, node_self_out, W_out):
    blk = 1000
    grid = N_NODES // blk
    return pl.pallas_call(
        _out_body,
        grid=(grid,),
        in_specs=[
            pl.BlockSpec((NC, blk, D_FEAT), lambda i: (0, i, 0)),
            pl.BlockSpec((blk, D_OUT), lambda i: (i, 0)),
            pl.BlockSpec((D_FEAT, D_OUT), lambda i: (0, 0)),
        ],
        out_specs=pl.BlockSpec((blk, D_OUT), lambda i: (i, 0)),
        out_shape=jax.ShapeDtypeStruct((N_NODES, D_OUT), jnp.float32),
    )(agg_pair, node_self_out, W_out)


@jax.jit
def kernel(node_input, edge_src, edge_dst, edge_attr, edge_scalar_attr,
           W_node, W1, W2, Wp, W_out):
    node_features, node_self_out = _node_linear(node_input, W_node)
    coeff = _radial(edge_scalar_attr, edge_attr, W1, W2, Wp)
    zeros = jnp.zeros((N_NODES, D_FEAT), jnp.float32)
    agg_pair = _sc_scatter(node_features, edge_src, edge_dst,
                           coeff.reshape(-1), zeros)
    return _out_linear(agg_pair, node_self_out, W_out)


# speedup vs baseline: 2.4079x; 2.0849x over previous
import functools

import jax
import jax.numpy as jnp
import numpy as np
from jax import lax
from jax.experimental import pallas as pl
from jax.experimental.pallas import tpu as pltpu
from jax.experimental.pallas import tpu_sc as plsc

N_NODES = 10000
N_EDGES = 320000
D_FEAT = 128
D_EDGE = 16
D_SC = 8
D_HID = 64
D_OUT = 128
MIXING = 0.15
NUM_NEIGHBORS = 32.0

NC = 2
NS = 16
NW = NC * NS
L = 16

EDGES_PER_TILE = N_EDGES // NW
CHUNK = 80
NCHUNKS = EDGES_PER_TILE // CHUNK
ROWS_PER_TILE = 624
TAIL_ROWS = N_NODES - NS * ROWS_PER_TILE


def _node_linear_body(x_ref, w_ref, feat_ref, self_ref):
    y = jnp.dot(x_ref[...], w_ref[...], preferred_element_type=jnp.float32)
    feat_ref[...] = y[:, :D_FEAT]
    self_ref[...] = y[:, D_FEAT:]


def _node_linear(node_input, W_node):
    blk = 1000
    grid = N_NODES // blk
    return pl.pallas_call(
        _node_linear_body,
        grid=(grid,),
        in_specs=[
            pl.BlockSpec((blk, D_FEAT), lambda i: (i, 0)),
            pl.BlockSpec((D_FEAT, D_FEAT + D_OUT), lambda i: (0, 0)),
        ],
        out_specs=[
            pl.BlockSpec((blk, D_FEAT), lambda i: (i, 0)),
            pl.BlockSpec((blk, D_OUT), lambda i: (i, 0)),
        ],
        out_shape=[
            jax.ShapeDtypeStruct((N_NODES, D_FEAT), jnp.float32),
            jax.ShapeDtypeStruct((N_NODES, D_OUT), jnp.float32),
        ],
    )(node_input, W_node)


def _radial_body(sc_ref, ea_ref, w1_ref, w2_ref, wp_ref, out_ref):
    h = jax.nn.gelu(jnp.dot(sc_ref[...], w1_ref[...], preferred_element_type=jnp.float32))
    h = jax.nn.gelu(jnp.dot(h, w2_ref[...], preferred_element_type=jnp.float32))
    w = jnp.dot(h, wp_ref[...], preferred_element_type=jnp.float32) * (1.0 / np.sqrt(D_HID))
    out_ref[...] = w * ea_ref[...]


def _radial(edge_scalar_attr, edge_attr, W1, W2, Wp):
    blk = 8000
    grid = N_EDGES // blk
    return pl.pallas_call(
        _radial_body,
        grid=(grid,),
        in_specs=[
            pl.BlockSpec((blk, D_SC), lambda i: (i, 0)),
            pl.BlockSpec((blk, D_EDGE), lambda i: (i, 0)),
            pl.BlockSpec((D_SC, D_HID), lambda i: (0, 0)),
            pl.BlockSpec((D_HID, D_HID), lambda i: (0, 0)),
            pl.BlockSpec((D_HID, D_EDGE), lambda i: (0, 0)),
        ],
        out_specs=pl.BlockSpec((blk, D_EDGE), lambda i: (i, 0)),
        out_shape=jax.ShapeDtypeStruct((N_EDGES, D_EDGE), jnp.float32),
    )(edge_scalar_attr, edge_attr, W1, W2, Wp)


def _sc_body(feat_hbm, src_hbm, dst_hbm, coeff_hbm, zeros_hbm, out_hbm,
             idx_v, dst_v, coeff_v, rows_v, msg_v, acc_shared):
    c = lax.axis_index("c")
    s = lax.axis_index("s")
    wid = c * NS + s

    pltpu.sync_copy(zeros_hbm.at[pl.ds(s * ROWS_PER_TILE, ROWS_PER_TILE)],
                    acc_shared.at[pl.ds(s * ROWS_PER_TILE, ROWS_PER_TILE)])

    @pl.when(s == NS - 1)
    def _():
        pltpu.sync_copy(
            zeros_hbm.at[pl.ds(NS * ROWS_PER_TILE, TAIL_ROWS)],
            acc_shared.at[pl.ds(NS * ROWS_PER_TILE, TAIL_ROWS)])

    plsc.subcore_barrier()

    tile_base = wid * EDGES_PER_TILE

    def chunk_body(ci, carry):
        base = tile_base + ci * CHUNK
        pltpu.sync_copy(src_hbm.at[pl.ds(base, CHUNK)], idx_v)
        pltpu.sync_copy(dst_hbm.at[pl.ds(base, CHUNK)], dst_v)
        pltpu.sync_copy(coeff_hbm.at[pl.ds(base, CHUNK)], coeff_v)
        pltpu.sync_copy(feat_hbm.at[idx_v], rows_v)

        lane = lax.iota(jnp.int32, L)
        half = lax.shift_right_logical(lane, 3)

        def edge_body(e, carry2):
            c16 = coeff_v[e, :]
            for v in range(D_FEAT // L):
                patt = 2 * v + half
                cexp = lax.gather(
                    c16, patt[:, None],
                    dimension_numbers=lax.GatherDimensionNumbers(
                        offset_dims=(), collapsed_slice_dims=(0,),
                        start_index_map=(0,)),
                    slice_sizes=(1,),
                    indices_are_sorted=True,
                    mode=lax.GatherScatterMode.PROMISE_IN_BOUNDS)
                row = rows_v[e, pl.ds(v * L, L)]
                msg_v[e, pl.ds(v * L, L)] = row * cexp
            return carry2

        lax.fori_loop(0, CHUNK, edge_body, 0, unroll=False)

        pltpu.sync_copy(msg_v, acc_shared.at[dst_v], add=True)
        return carry

    lax.fori_loop(0, NCHUNKS, chunk_body, 0, unroll=False)

    plsc.subcore_barrier()
    pltpu.sync_copy(acc_shared.at[pl.ds(s * ROWS_PER_TILE, ROWS_PER_TILE)],
                    out_hbm.at[c, pl.ds(s * ROWS_PER_TILE, ROWS_PER_TILE)])

    @pl.when(s == NS - 1)
    def _():
        pltpu.sync_copy(
            acc_shared.at[pl.ds(NS * ROWS_PER_TILE, TAIL_ROWS)],
            out_hbm.at[c, pl.ds(NS * ROWS_PER_TILE, TAIL_ROWS)])


def _sc_scatter(node_features, edge_src, edge_dst, coeff, zeros):
    mesh = plsc.VectorSubcoreMesh(core_axis_name="c", subcore_axis_name="s",
                                  num_cores=NC, num_subcores=NS)
    kern = pl.kernel(
        _sc_body,
        out_type=jax.ShapeDtypeStruct((NC, N_NODES, D_FEAT), jnp.float32),
        mesh=mesh,
        compiler_params=pltpu.CompilerParams(needs_layout_passes=False),
        scratch_types=[
            pltpu.VMEM((CHUNK,), jnp.int32),
            pltpu.VMEM((CHUNK,), jnp.int32),
            pltpu.VMEM((CHUNK, D_EDGE), jnp.float32),
            pltpu.VMEM((CHUNK, D_FEAT), jnp.float32),
            pltpu.VMEM((CHUNK, D_FEAT), jnp.float32),
            pltpu.VMEM_SHARED((N_NODES, D_FEAT), jnp.float32),
        ],
    )
    return kern(node_features, edge_src, edge_dst, coeff, zeros)


def _out_body(agg_ref, self_ref, wout_ref, out_ref):
    c_self = np.float32(np.sqrt(1.0 - MIXING))
    c_conv = np.float32(np.sqrt(MIXING) / np.sqrt(NUM_NEIGHBORS))
    agg = agg_ref[0] + agg_ref[1]
    conv = jnp.dot(agg, wout_ref[...], preferred_element_type=jnp.float32)
    out_ref[...] = c_self * self_ref[...] + c_conv * conv


def _out_linear(agg_pair, node_self_out, W_out):
    blk = 1000
    grid = N_NODES // blk
    return pl.pallas_call(
        _out_body,
        grid=(grid,),
        in_specs=[
            pl.BlockSpec((NC, blk, D_FEAT), lambda i: (0, i, 0)),
            pl.BlockSpec((blk, D_OUT), lambda i: (i, 0)),
            pl.BlockSpec((D_FEAT, D_OUT), lambda i: (0, 0)),
        ],
        out_specs=pl.BlockSpec((blk, D_OUT), lambda i: (i, 0)),
        out_shape=jax.ShapeDtypeStruct((N_NODES, D_OUT), jnp.float32),
    )(agg_pair, node_self_out, W_out)


@jax.jit
def kernel(node_input, edge_src, edge_dst, edge_attr, edge_scalar_attr,
           W_node, W1, W2, Wp, W_out):
    node_features, node_self_out = _node_linear(node_input, W_node)
    coeff = _radial(edge_scalar_attr, edge_attr, W1, W2, Wp)
    zeros = jnp.zeros((N_NODES, D_FEAT), jnp.float32)
    agg_pair = _sc_scatter(node_features, edge_src, edge_dst, coeff, zeros)
    return _out_linear(agg_pair, node_self_out, W_out)


# speedup vs baseline: 3.1471x; 1.3070x over previous
import functools

import jax
import jax.numpy as jnp
import numpy as np
from jax import lax
from jax.experimental import pallas as pl
from jax.experimental.pallas import tpu as pltpu
from jax.experimental.pallas import tpu_sc as plsc

N_NODES = 10000
N_EDGES = 320000
D_FEAT = 128
D_EDGE = 16
D_SC = 8
D_HID = 64
D_OUT = 128
MIXING = 0.15
NUM_NEIGHBORS = 32.0

NC = 2
NS = 16
NW = NC * NS
L = 16
NV = D_FEAT // L

EDGES_PER_TILE = N_EDGES // NW
CHUNK = 80
NCHUNKS = EDGES_PER_TILE // CHUNK


def _node_linear_body(x_ref, w_ref, feat_ref, self_ref):
    y = jnp.dot(x_ref[...], w_ref[...], preferred_element_type=jnp.float32)
    feat_ref[...] = y[:, :D_FEAT]
    self_ref[...] = y[:, D_FEAT:]


def _node_linear(node_input, W_node):
    blk = 1000
    grid = N_NODES // blk
    return pl.pallas_call(
        _node_linear_body,
        grid=(grid,),
        in_specs=[
            pl.BlockSpec((blk, D_FEAT), lambda i: (i, 0)),
            pl.BlockSpec((D_FEAT, D_FEAT + D_OUT), lambda i: (0, 0)),
        ],
        out_specs=[
            pl.BlockSpec((blk, D_FEAT), lambda i: (i, 0)),
            pl.BlockSpec((blk, D_OUT), lambda i: (i, 0)),
        ],
        out_shape=[
            jax.ShapeDtypeStruct((N_NODES, D_FEAT), jnp.float32),
            jax.ShapeDtypeStruct((N_NODES, D_OUT), jnp.float32),
        ],
    )(node_input, W_node)


def _radial_body(sc_ref, ea_ref, w1_ref, w2_ref, wp_ref, out_ref):
    h = jax.nn.gelu(jnp.dot(sc_ref[...], w1_ref[...],
                            preferred_element_type=jnp.float32))
    h = jax.nn.gelu(jnp.dot(h, w2_ref[...],
                            preferred_element_type=jnp.float32))
    w = jnp.dot(h, wp_ref[...],
                preferred_element_type=jnp.float32) * (1.0 / np.sqrt(D_HID))
    out_ref[...] = w * ea_ref[...]


def _radial(edge_scalar_attr, edge_attr, W1, W2, Wp):
    blk = 8000
    grid = N_EDGES // blk
    return pl.pallas_call(
        _radial_body,
        grid=(grid,),
        in_specs=[
            pl.BlockSpec((blk, D_SC), lambda i: (i, 0)),
            pl.BlockSpec((blk, D_EDGE), lambda i: (i, 0)),
            pl.BlockSpec((D_SC, D_HID), lambda i: (0, 0)),
            pl.BlockSpec((D_HID, D_HID), lambda i: (0, 0)),
            pl.BlockSpec((D_HID, D_EDGE), lambda i: (0, 0)),
        ],
        out_specs=pl.BlockSpec((blk, D_EDGE), lambda i: (i, 0)),
        out_shape=jax.ShapeDtypeStruct((N_EDGES, D_EDGE), jnp.float32),
    )(edge_scalar_attr, edge_attr, W1, W2, Wp)


def _sc_body(feat_hbm, src_hbm, dst_hbm, coeff_hbm,
             out_hbm, bndrow_hbm, bndidx_hbm,
             src4, dst4, coeff4, rows2, stg_rows, stg_idx, bnd_buf, zpeek,
             sem_lin, sem_gath):
    c = lax.axis_index("c")
    s = lax.axis_index("s")
    wid = c * NS + s
    tile_base = wid * EDGES_PER_TILE

    lane = lax.iota(jnp.int32, L)
    zerov = jnp.zeros((L,), jnp.float32)
    lane0 = lane == 0

    def store_idx_scalar(pos, val):
        plsc.store_scatter(stg_idx, [jnp.broadcast_to(pos, (L,))],
                           jnp.broadcast_to(val, (L,)), mask=lane0)

    def lin_start(ci):
        base = tile_base + ci * CHUNK
        s4 = lax.rem(ci, 4)
        pltpu.async_copy(src_hbm.at[pl.ds(base, CHUNK)], src4.at[s4],
                         sem_lin.at[s4])
        pltpu.async_copy(dst_hbm.at[pl.ds(base, CHUNK)], dst4.at[s4],
                         sem_lin.at[s4])
        pltpu.async_copy(coeff_hbm.at[pl.ds(base, CHUNK)], coeff4.at[s4],
                         sem_lin.at[s4])

    def lin_wait(ci):
        base = tile_base + ci * CHUNK
        s4 = lax.rem(ci, 4)
        pltpu.make_async_copy(src_hbm.at[pl.ds(base, CHUNK)], src4.at[s4],
                              sem_lin.at[s4]).wait()
        pltpu.make_async_copy(dst_hbm.at[pl.ds(base, CHUNK)], dst4.at[s4],
                              sem_lin.at[s4]).wait()
        pltpu.make_async_copy(coeff_hbm.at[pl.ds(base, CHUNK)], coeff4.at[s4],
                              sem_lin.at[s4]).wait()

    def gather_start(ci):
        s4 = lax.rem(ci, 4)
        s2 = lax.rem(ci, 2)
        pltpu.async_copy(feat_hbm.at[src4.at[s4]], rows2.at[s2],
                         sem_gath.at[s2])

    def gather_wait(ci):
        s4 = lax.rem(ci, 4)
        s2 = lax.rem(ci, 2)
        pltpu.make_async_copy(feat_hbm.at[src4.at[s4]], rows2.at[s2],
                              sem_gath.at[s2]).wait()

    lin_start(0)
    lin_start(1)
    lin_wait(0)
    gather_start(0)

    @pl.when(wid < NW - 1)
    def _():
        pltpu.sync_copy(dst_hbm.at[pl.ds(tile_base + EDGES_PER_TILE, L)],
                        zpeek)

    first_dst = dst4[0, pl.ds(0, L)][0]
    zstart = jnp.where(wid == 0, 0, first_dst)
    zend = jnp.where(wid == NW - 1, N_NODES, zpeek[...][0])

    def zero_stage_row(j, carry):
        for v in range(NV):
            stg_rows[j, pl.ds(v * L, L)] = zerov
        return carry

    lax.fori_loop(0, CHUNK, zero_stage_row, 0, unroll=False)
    for v in range(2 * NV):
        bnd_buf[pl.ds(v * L, L)] = zerov

    def zero_shot(q, carry):
        base_row = zstart + q * CHUNK

        @pl.when(base_row < zend)
        def _():
            for v in range(CHUNK // L):
                vals = jnp.minimum(base_row + v * L + lane, zend - 1)
                stg_idx[pl.ds(v * L, L)] = vals
            pltpu.sync_copy(stg_rows, out_hbm.at[stg_idx])
        return carry

    lax.fori_loop(0, (N_NODES + CHUNK - 1) // CHUNK, zero_shot, 0,
                  unroll=False)

    half = lax.shift_right_logical(lane, 3)
    gdn = lax.GatherDimensionNumbers(offset_dims=(), collapsed_slice_dims=(0,),
                                     start_index_map=(0,))

    def edge_step(ci, e, d, carry):
        cur, run_idx, fill, bfirst, last_staged, acc = carry
        s4 = lax.rem(ci, 4)
        s2 = lax.rem(ci, 2)
        changed = d != cur

        @pl.when(changed)
        def _():
            @pl.when(run_idx == 0)
            def _():
                for v in range(NV):
                    bnd_buf[pl.ds(v * L, L)] = acc[v]

            @pl.when(run_idx > 0)
            def _():
                for v in range(NV):
                    stg_rows[fill, pl.ds(v * L, L)] = acc[v]
                store_idx_scalar(fill, cur)

        bfirst = jnp.where(changed & (run_idx == 0), cur, bfirst)
        last_staged = jnp.where(changed & (run_idx > 0), cur, last_staged)
        fill = fill + jnp.where(changed & (run_idx > 0), 1, 0)
        run_idx = run_idx + jnp.where(changed, 1, 0)

        @pl.when(fill >= CHUNK)
        def _():
            pltpu.sync_copy(stg_rows, out_hbm.at[stg_idx])

        fill = jnp.where(fill >= CHUNK, 0, fill)
        cur = jnp.where(changed, d, cur)

        c16 = coeff4[s4, e, :]
        gate = jnp.where(changed, 0.0, 1.0)
        new_acc = []
        for v in range(NV):
            patt = 2 * v + half
            cexp = lax.gather(
                c16, patt[:, None], dimension_numbers=gdn,
                slice_sizes=(1,), indices_are_sorted=True,
                mode=lax.GatherScatterMode.PROMISE_IN_BOUNDS)
            row = rows2[s2, e, pl.ds(v * L, L)]
            new_acc.append(row * cexp + acc[v] * gate)
        return (cur, run_idx, fill, bfirst, last_staged, tuple(new_acc))

    def group_step(ci, g, carry):
        s4 = lax.rem(ci, 4)
        dvec = dst4[s4, pl.ds(g * L, L)]
        for k in range(L):
            carry = edge_step(ci, g * L + k, dvec[k], carry)
        return carry

    def chunk_body(ci, carry):
        @pl.when(ci + 2 < NCHUNKS)
        def _():
            lin_start(ci + 2)

        @pl.when(ci + 1 < NCHUNKS)
        def _():
            lin_wait(ci + 1)
            gather_start(ci + 1)

        gather_wait(ci)
        return lax.fori_loop(0, CHUNK // L,
                             lambda g, cc: group_step(ci, g, cc),
                             carry, unroll=False)

    acc0 = tuple(zerov for _ in range(NV))
    init = (first_dst, jnp.int32(0), jnp.int32(0), jnp.int32(0), jnp.int32(0),
            acc0)
    cur, run_idx, fill, bfirst, last_staged, acc = lax.fori_loop(
        0, NCHUNKS, chunk_body, init, unroll=False)

    @pl.when(fill > 0)
    def _():
        lastrow = fill - 1
        lastidx = last_staged

        def pad_row(j, carry):
            @pl.when(j >= fill)
            def _():
                for v in range(NV):
                    stg_rows[j, pl.ds(v * L, L)] = \
                        stg_rows[lastrow, pl.ds(v * L, L)]
                store_idx_scalar(j, lastidx)
            return carry

        lax.fori_loop(0, CHUNK, pad_row, 0, unroll=False)
        pltpu.sync_copy(stg_rows, out_hbm.at[stg_idx])

    for v in range(NV):
        bnd_buf[pl.ds(NV * L + v * L, L)] = acc[v]
    pltpu.sync_copy(bnd_buf, bndrow_hbm.at[pl.ds(wid * 2 * D_FEAT,
                                                 2 * D_FEAT)])

    bvec = jnp.where(lane == 0, bfirst, jnp.where(lane == 1, cur, 0))
    stg_idx[pl.ds(0, L)] = bvec
    pltpu.sync_copy(stg_idx.at[pl.ds(0, L)], bndidx_hbm.at[pl.ds(wid * L, L)])


def _sc_scatter(node_features, edge_src, edge_dst, coeff):
    mesh = plsc.VectorSubcoreMesh(core_axis_name="c", subcore_axis_name="s",
                                  num_cores=NC, num_subcores=NS)
    kern = pl.kernel(
        _sc_body,
        out_type=[
            jax.ShapeDtypeStruct((N_NODES, D_FEAT), jnp.float32),
            jax.ShapeDtypeStruct((NW * 2 * D_FEAT,), jnp.float32),
            jax.ShapeDtypeStruct((NW * L,), jnp.int32),
        ],
        mesh=mesh,
        compiler_params=pltpu.CompilerParams(needs_layout_passes=False),
        scratch_types=[
            pltpu.VMEM((4, CHUNK), jnp.int32),
            pltpu.VMEM((4, CHUNK), jnp.int32),
            pltpu.VMEM((4, CHUNK, D_EDGE), jnp.float32),
            pltpu.VMEM((2, CHUNK, D_FEAT), jnp.float32),
            pltpu.VMEM((CHUNK, D_FEAT), jnp.float32),
            pltpu.VMEM((CHUNK,), jnp.int32),
            pltpu.VMEM((2 * D_FEAT,), jnp.float32),
            pltpu.VMEM((L,), jnp.int32),
            pltpu.SemaphoreType.DMA((4,)),
            pltpu.SemaphoreType.DMA((2,)),
        ],
    )
    return kern(node_features, edge_src, edge_dst, coeff)


def _out_body(agg_ref, bndrow_ref, bndidx_ref, self_ref, wout_ref, out_ref,
              *, blk):
    i = pl.program_id(0)
    c_self = np.float32(np.sqrt(1.0 - MIXING))
    c_conv = np.float32(np.sqrt(MIXING) / np.sqrt(NUM_NEIGHBORS))
    rows = lax.broadcasted_iota(jnp.int32, (blk, 2 * NW), 0) + i * blk
    onehot = (rows == bndidx_ref[...].reshape(1, 2 * NW)).astype(jnp.float32)
    fix = jnp.dot(onehot, bndrow_ref[...], preferred_element_type=jnp.float32)
    agg = agg_ref[...] + fix
    conv = jnp.dot(agg, wout_ref[...], preferred_element_type=jnp.float32)
    out_ref[...] = c_self * self_ref[...] + c_conv * conv


def _out_linear(agg, bnd_rows, bnd_idx, node_self_out, W_out):
    blk = 1000
    grid = N_NODES // blk
    return pl.pallas_call(
        functools.partial(_out_body, blk=blk),
        grid=(grid,),
        in_specs=[
            pl.BlockSpec((blk, D_FEAT), lambda i: (i, 0)),
            pl.BlockSpec((2 * NW, D_FEAT), lambda i: (0, 0)),
            pl.BlockSpec((1, 2 * NW), lambda i: (0, 0)),
            pl.BlockSpec((blk, D_OUT), lambda i: (i, 0)),
            pl.BlockSpec((D_FEAT, D_OUT), lambda i: (0, 0)),
        ],
        out_specs=pl.BlockSpec((blk, D_OUT), lambda i: (i, 0)),
        out_shape=jax.ShapeDtypeStruct((N_NODES, D_OUT), jnp.float32),
    )(agg, bnd_rows, bnd_idx, node_self_out, W_out)


@jax.jit
def kernel(node_input, edge_src, edge_dst, edge_attr, edge_scalar_attr,
           W_node, W1, W2, Wp, W_out):
    node_features, node_self_out = _node_linear(node_input, W_node)
    coeff = _radial(edge_scalar_attr, edge_attr, W1, W2, Wp)
    agg, bnd_pair, bnd_idx = _sc_scatter(node_features, edge_src, edge_dst,
                                         coeff)
    bnd_rows = bnd_pair.reshape(NW * 2, D_FEAT)
    bidx = bnd_idx.reshape(NW, L)[:, :2].reshape(1, 2 * NW)
    return _out_linear(agg, bnd_rows, bidx, node_self_out, W_out)


# speedup vs baseline: 4.4237x; 1.4056x over previous
import functools

import jax
import jax.numpy as jnp
import numpy as np
from jax import lax
from jax.experimental import pallas as pl
from jax.experimental.pallas import tpu as pltpu
from jax.experimental.pallas import tpu_sc as plsc

N_NODES = 10000
N_EDGES = 320000
D_FEAT = 128
D_EDGE = 16
D_SC = 8
D_HID = 64
D_OUT = 128
MIXING = 0.15
NUM_NEIGHBORS = 32.0

NC = 2
NS = 16
NW = NC * NS
L = 16
NV = D_FEAT // L

EDGES_PER_TILE = N_EDGES // NW
CHUNK = 80
NCHUNKS = EDGES_PER_TILE // CHUNK


def _node_linear_body(x_ref, w_ref, feat_ref, self_ref):
    y = jnp.dot(x_ref[...], w_ref[...], preferred_element_type=jnp.float32)
    feat_ref[...] = y[:, :D_FEAT]
    self_ref[...] = y[:, D_FEAT:]


def _node_linear(node_input, W_node):
    blk = 1000
    grid = N_NODES // blk
    return pl.pallas_call(
        _node_linear_body,
        grid=(grid,),
        in_specs=[
            pl.BlockSpec((blk, D_FEAT), lambda i: (i, 0)),
            pl.BlockSpec((D_FEAT, D_FEAT + D_OUT), lambda i: (0, 0)),
        ],
        out_specs=[
            pl.BlockSpec((blk, D_FEAT), lambda i: (i, 0)),
            pl.BlockSpec((blk, D_OUT), lambda i: (i, 0)),
        ],
        out_shape=[
            jax.ShapeDtypeStruct((N_NODES, D_FEAT), jnp.float32),
            jax.ShapeDtypeStruct((N_NODES, D_OUT), jnp.float32),
        ],
    )(node_input, W_node)


def _radial_body(sct_ref, eat_ref, w1_ref, w2_ref, wp_ref, out_ref):
    c00 = (((0,), (0,)), ((), ()))
    xb = sct_ref[...].astype(jnp.bfloat16)
    w1b = w1_ref[...].astype(jnp.bfloat16)
    w2b = w2_ref[...].astype(jnp.bfloat16)
    wpb = wp_ref[...].astype(jnp.bfloat16)
    h = jax.nn.gelu(lax.dot_general(w1b, xb, c00,
                                    preferred_element_type=jnp.float32))
    h2 = jax.nn.gelu(lax.dot_general(w2b, h.astype(jnp.bfloat16), c00,
                                     preferred_element_type=jnp.float32))
    w = lax.dot_general(h2.astype(jnp.bfloat16), wpb, c00,
                        preferred_element_type=jnp.float32)
    eye = (lax.broadcasted_iota(jnp.int32, (D_EDGE, D_EDGE), 0) ==
           lax.broadcasted_iota(jnp.int32, (D_EDGE, D_EDGE), 1)
           ).astype(jnp.float32)
    ea = lax.dot_general(eat_ref[...], eye, c00,
                         preferred_element_type=jnp.float32)
    out_ref[...] = w * (1.0 / np.sqrt(D_HID)) * ea


def _radial(edge_scalar_attr_t, edge_attr_t, W1, W2, Wp):
    blk = 6400
    grid = N_EDGES // blk
    return pl.pallas_call(
        _radial_body,
        grid=(grid,),
        in_specs=[
            pl.BlockSpec((D_SC, blk), lambda i: (0, i)),
            pl.BlockSpec((D_EDGE, blk), lambda i: (0, i)),
            pl.BlockSpec((D_SC, D_HID), lambda i: (0, 0)),
            pl.BlockSpec((D_HID, D_HID), lambda i: (0, 0)),
            pl.BlockSpec((D_HID, D_EDGE), lambda i: (0, 0)),
        ],
        out_specs=pl.BlockSpec((blk, D_EDGE), lambda i: (i, 0)),
        out_shape=jax.ShapeDtypeStruct((N_EDGES, D_EDGE), jnp.float32),
    )(edge_scalar_attr_t, edge_attr_t, W1, W2, Wp)


def _sc_body(feat_hbm, src_hbm, dst_hbm, coeff_hbm,
             out_hbm, bndrow_hbm, bndidx_hbm,
             src4, dst4, coeff4, rows2, stg_rows, stg_idx, bnd_buf, zpeek,
             sem_lin, sem_gath):
    c = lax.axis_index("c")
    s = lax.axis_index("s")
    wid = c * NS + s
    tile_base = wid * EDGES_PER_TILE

    lane = lax.iota(jnp.int32, L)
    zerov = jnp.zeros((L,), jnp.float32)
    lane0 = lane == 0

    def store_idx_scalar(pos, val):
        plsc.store_scatter(stg_idx, [jnp.broadcast_to(pos, (L,))],
                           jnp.broadcast_to(val, (L,)), mask=lane0)

    def lin_start(ci):
        base = tile_base + ci * CHUNK
        s4 = lax.rem(ci, 4)
        pltpu.async_copy(src_hbm.at[pl.ds(base, CHUNK)], src4.at[s4],
                         sem_lin.at[s4])
        pltpu.async_copy(dst_hbm.at[pl.ds(base, CHUNK)], dst4.at[s4],
                         sem_lin.at[s4])
        pltpu.async_copy(coeff_hbm.at[pl.ds(base, CHUNK)], coeff4.at[s4],
                         sem_lin.at[s4])

    def lin_wait(ci):
        base = tile_base + ci * CHUNK
        s4 = lax.rem(ci, 4)
        pltpu.make_async_copy(src_hbm.at[pl.ds(base, CHUNK)], src4.at[s4],
                              sem_lin.at[s4]).wait()
        pltpu.make_async_copy(dst_hbm.at[pl.ds(base, CHUNK)], dst4.at[s4],
                              sem_lin.at[s4]).wait()
        pltpu.make_async_copy(coeff_hbm.at[pl.ds(base, CHUNK)], coeff4.at[s4],
                              sem_lin.at[s4]).wait()

    def gather_start(ci):
        s4 = lax.rem(ci, 4)
        s2 = lax.rem(ci, 2)
        pltpu.async_copy(feat_hbm.at[src4.at[s4]], rows2.at[s2],
                         sem_gath.at[s2])

    def gather_wait(ci):
        s4 = lax.rem(ci, 4)
        s2 = lax.rem(ci, 2)
        pltpu.make_async_copy(feat_hbm.at[src4.at[s4]], rows2.at[s2],
                              sem_gath.at[s2]).wait()

    lin_start(0)
    lin_start(1)
    lin_wait(0)
    gather_start(0)

    @pl.when(wid < NW - 1)
    def _():
        pltpu.sync_copy(dst_hbm.at[pl.ds(tile_base + EDGES_PER_TILE, L)],
                        zpeek)

    first_dst = dst4[0, pl.ds(0, L)][0]
    zstart = jnp.where(wid == 0, 0, first_dst)
    zend = jnp.where(wid == NW - 1, N_NODES, zpeek[...][0])

    def zero_stage_row(j, carry):
        for v in range(NV):
            stg_rows[j, pl.ds(v * L, L)] = zerov
        return carry

    lax.fori_loop(0, CHUNK, zero_stage_row, 0, unroll=False)
    for v in range(2 * NV):
        bnd_buf[pl.ds(v * L, L)] = zerov

    def zero_shot(q, carry):
        base_row = zstart + q * CHUNK

        @pl.when(base_row < zend)
        def _():
            for v in range(CHUNK // L):
                vals = jnp.minimum(base_row + v * L + lane, zend - 1)
                stg_idx[pl.ds(v * L, L)] = vals
            pltpu.sync_copy(stg_rows, out_hbm.at[stg_idx])
        return carry

    lax.fori_loop(0, (N_NODES + CHUNK - 1) // CHUNK, zero_shot, 0,
                  unroll=False)

    half = lax.shift_right_logical(lane, 3)
    gdn = lax.GatherDimensionNumbers(offset_dims=(), collapsed_slice_dims=(0,),
                                     start_index_map=(0,))

    def edge_step(ci, e, d, carry):
        cur, run_idx, fill, bfirst, last_staged, acc = carry
        s4 = lax.rem(ci, 4)
        s2 = lax.rem(ci, 2)
        changed = d != cur

        @pl.when(changed)
        def _():
            @pl.when(run_idx == 0)
            def _():
                for v in range(NV):
                    bnd_buf[pl.ds(v * L, L)] = acc[v]

            @pl.when(run_idx > 0)
            def _():
                for v in range(NV):
                    stg_rows[fill, pl.ds(v * L, L)] = acc[v]
                store_idx_scalar(fill, cur)

        bfirst = jnp.where(changed & (run_idx == 0), cur, bfirst)
        last_staged = jnp.where(changed & (run_idx > 0), cur, last_staged)
        fill = fill + jnp.where(changed & (run_idx > 0), 1, 0)
        run_idx = run_idx + jnp.where(changed, 1, 0)

        @pl.when(fill >= CHUNK)
        def _():
            pltpu.sync_copy(stg_rows, out_hbm.at[stg_idx])

        fill = jnp.where(fill >= CHUNK, 0, fill)
        cur = jnp.where(changed, d, cur)

        c16 = coeff4[s4, e, :]
        gate = jnp.where(changed, 0.0, 1.0)
        new_acc = []
        for v in range(NV):
            patt = 2 * v + half
            cexp = lax.gather(
                c16, patt[:, None], dimension_numbers=gdn,
                slice_sizes=(1,), indices_are_sorted=True,
                mode=lax.GatherScatterMode.PROMISE_IN_BOUNDS)
            row = rows2[s2, e, pl.ds(v * L, L)]
            new_acc.append(row * cexp + acc[v] * gate)
        return (cur, run_idx, fill, bfirst, last_staged, tuple(new_acc))

    def group_step(ci, g, carry):
        s4 = lax.rem(ci, 4)
        dvec = dst4[s4, pl.ds(g * L, L)]
        for k in range(L):
            carry = edge_step(ci, g * L + k, dvec[k], carry)
        return carry

    def chunk_body(ci, carry):
        @pl.when(ci + 2 < NCHUNKS)
        def _():
            lin_start(ci + 2)

        @pl.when(ci + 1 < NCHUNKS)
        def _():
            lin_wait(ci + 1)
            gather_start(ci + 1)

        gather_wait(ci)
        return lax.fori_loop(0, CHUNK // L,
                             lambda g, cc: group_step(ci, g, cc),
                             carry, unroll=False)

    acc0 = tuple(zerov for _ in range(NV))
    init = (first_dst, jnp.int32(0), jnp.int32(0), jnp.int32(0), jnp.int32(0),
            acc0)
    cur, run_idx, fill, bfirst, last_staged, acc = lax.fori_loop(
        0, NCHUNKS, chunk_body, init, unroll=False)

    @pl.when(fill > 0)
    def _():
        lastrow = fill - 1
        lastidx = last_staged

        def pad_row(j, carry):
            @pl.when(j >= fill)
            def _():
                for v in range(NV):
                    stg_rows[j, pl.ds(v * L, L)] = \
                        stg_rows[lastrow, pl.ds(v * L, L)]
                store_idx_scalar(j, lastidx)
            return carry

        lax.fori_loop(0, CHUNK, pad_row, 0, unroll=False)
        pltpu.sync_copy(stg_rows, out_hbm.at[stg_idx])

    for v in range(NV):
        bnd_buf[pl.ds(NV * L + v * L, L)] = acc[v]
    pltpu.sync_copy(bnd_buf, bndrow_hbm.at[pl.ds(wid * 2 * D_FEAT,
                                                 2 * D_FEAT)])

    bvec = jnp.where(lane == 0, bfirst, jnp.where(lane == 1, cur, 0))
    stg_idx[pl.ds(0, L)] = bvec
    pltpu.sync_copy(stg_idx.at[pl.ds(0, L)], bndidx_hbm.at[pl.ds(wid * L, L)])


def _sc_scatter(node_features, edge_src, edge_dst, coeff):
    mesh = plsc.VectorSubcoreMesh(core_axis_name="c", subcore_axis_name="s",
                                  num_cores=NC, num_subcores=NS)
    kern = pl.kernel(
        _sc_body,
        out_type=[
            jax.ShapeDtypeStruct((N_NODES, D_FEAT), jnp.float32),
            jax.ShapeDtypeStruct((NW * 2 * D_FEAT,), jnp.float32),
            jax.ShapeDtypeStruct((NW * L,), jnp.int32),
        ],
        mesh=mesh,
        compiler_params=pltpu.CompilerParams(needs_layout_passes=False),
        scratch_types=[
            pltpu.VMEM((4, CHUNK), jnp.int32),
            pltpu.VMEM((4, CHUNK), jnp.int32),
            pltpu.VMEM((4, CHUNK, D_EDGE), jnp.float32),
            pltpu.VMEM((2, CHUNK, D_FEAT), jnp.float32),
            pltpu.VMEM((CHUNK, D_FEAT), jnp.float32),
            pltpu.VMEM((CHUNK,), jnp.int32),
            pltpu.VMEM((2 * D_FEAT,), jnp.float32),
            pltpu.VMEM((L,), jnp.int32),
            pltpu.SemaphoreType.DMA((4,)),
            pltpu.SemaphoreType.DMA((2,)),
        ],
    )
    return kern(node_features, edge_src, edge_dst, coeff)


def _out_body(agg_ref, bndrow_ref, bndidx_ref, self_ref, wout_ref, out_ref,
              *, blk):
    i = pl.program_id(0)
    c_self = np.float32(np.sqrt(1.0 - MIXING))
    c_conv = np.float32(np.sqrt(MIXING) / np.sqrt(NUM_NEIGHBORS))
    rows = lax.broadcasted_iota(jnp.int32, (blk, 2 * NW), 0) + i * blk
    onehot = (rows == bndidx_ref[...].reshape(1, 2 * NW)).astype(jnp.float32)
    fix = jnp.dot(onehot, bndrow_ref[...], preferred_element_type=jnp.float32)
    agg = agg_ref[...] + fix
    conv = jnp.dot(agg, wout_ref[...], preferred_element_type=jnp.float32)
    out_ref[...] = c_self * self_ref[...] + c_conv * conv


def _out_linear(agg, bnd_rows, bnd_idx, node_self_out, W_out):
    blk = 1000
    grid = N_NODES // blk
    return pl.pallas_call(
        functools.partial(_out_body, blk=blk),
        grid=(grid,),
        in_specs=[
            pl.BlockSpec((blk, D_FEAT), lambda i: (i, 0)),
            pl.BlockSpec((2 * NW, D_FEAT), lambda i: (0, 0)),
            pl.BlockSpec((1, 2 * NW), lambda i: (0, 0)),
            pl.BlockSpec((blk, D_OUT), lambda i: (i, 0)),
            pl.BlockSpec((D_FEAT, D_OUT), lambda i: (0, 0)),
        ],
        out_specs=pl.BlockSpec((blk, D_OUT), lambda i: (i, 0)),
        out_shape=jax.ShapeDtypeStruct((N_NODES, D_OUT), jnp.float32),
    )(agg, bnd_rows, bnd_idx, node_self_out, W_out)


@jax.jit
def kernel(node_input, edge_src, edge_dst, edge_attr, edge_scalar_attr,
           W_node, W1, W2, Wp, W_out):
    node_features, node_self_out = _node_linear(node_input, W_node)
    coeff = _radial(edge_scalar_attr.T, edge_attr.T, W1, W2, Wp)
    agg, bnd_pair, bnd_idx = _sc_scatter(node_features, edge_src, edge_dst,
                                         coeff)
    bnd_rows = bnd_pair.reshape(NW * 2, D_FEAT)
    bidx = bnd_idx.reshape(NW, L)[:, :2].reshape(1, 2 * NW)
    return _out_linear(agg, bnd_rows, bidx, node_self_out, W_out)


# speedup vs baseline: 4.9865x; 1.1272x over previous
import functools

import jax
import jax.numpy as jnp
import numpy as np
from jax import lax
from jax.experimental import pallas as pl
from jax.experimental.pallas import tpu as pltpu
from jax.experimental.pallas import tpu_sc as plsc

N_NODES = 10000
N_EDGES = 320000
D_FEAT = 128
D_EDGE = 16
D_SC = 8
D_HID = 64
D_OUT = 128
MIXING = 0.15
NUM_NEIGHBORS = 32.0

NC = 2
NS = 16
NW = NC * NS
L = 16
NV = D_FEAT // L

EDGES_PER_TILE = N_EDGES // NW
CHUNK = 80
NCHUNKS = EDGES_PER_TILE // CHUNK
STG = 96


def _node_linear_body(x_ref, w_ref, feat_ref, self_ref):
    y = jnp.dot(x_ref[...], w_ref[...], preferred_element_type=jnp.float32)
    feat_ref[...] = y[:, :D_FEAT]
    self_ref[...] = y[:, D_FEAT:]


def _node_linear(node_input, W_node):
    blk = 1000
    grid = N_NODES // blk
    return pl.pallas_call(
        _node_linear_body,
        grid=(grid,),
        in_specs=[
            pl.BlockSpec((blk, D_FEAT), lambda i: (i, 0)),
            pl.BlockSpec((D_FEAT, D_FEAT + D_OUT), lambda i: (0, 0)),
        ],
        out_specs=[
            pl.BlockSpec((blk, D_FEAT), lambda i: (i, 0)),
            pl.BlockSpec((blk, D_OUT), lambda i: (i, 0)),
        ],
        out_shape=[
            jax.ShapeDtypeStruct((N_NODES, D_FEAT), jnp.float32),
            jax.ShapeDtypeStruct((N_NODES, D_OUT), jnp.float32),
        ],
    )(node_input, W_node)


def _radial_body(sct_ref, eat_ref, w1_ref, w2_ref, wp_ref, out_ref):
    c00 = (((0,), (0,)), ((), ()))
    xb = sct_ref[...].astype(jnp.bfloat16)
    w1b = w1_ref[...].astype(jnp.bfloat16)
    w2b = w2_ref[...].astype(jnp.bfloat16)
    wpb = wp_ref[...].astype(jnp.bfloat16)
    h = jax.nn.gelu(lax.dot_general(w1b, xb, c00,
                                    preferred_element_type=jnp.float32))
    h2 = jax.nn.gelu(lax.dot_general(w2b, h.astype(jnp.bfloat16), c00,
                                     preferred_element_type=jnp.float32))
    w = lax.dot_general(h2.astype(jnp.bfloat16), wpb, c00,
                        preferred_element_type=jnp.float32)
    eye = (lax.broadcasted_iota(jnp.int32, (D_EDGE, D_EDGE), 0) ==
           lax.broadcasted_iota(jnp.int32, (D_EDGE, D_EDGE), 1)
           ).astype(jnp.float32)
    ea = lax.dot_general(eat_ref[...], eye, c00,
                         preferred_element_type=jnp.float32)
    out_ref[...] = w * (1.0 / np.sqrt(D_HID)) * ea


def _radial(edge_scalar_attr_t, edge_attr_t, W1, W2, Wp):
    blk = 6400
    grid = N_EDGES // blk
    return pl.pallas_call(
        _radial_body,
        grid=(grid,),
        in_specs=[
            pl.BlockSpec((D_SC, blk), lambda i: (0, i)),
            pl.BlockSpec((D_EDGE, blk), lambda i: (0, i)),
            pl.BlockSpec((D_SC, D_HID), lambda i: (0, 0)),
            pl.BlockSpec((D_HID, D_HID), lambda i: (0, 0)),
            pl.BlockSpec((D_HID, D_EDGE), lambda i: (0, 0)),
        ],
        out_specs=pl.BlockSpec((blk, D_EDGE), lambda i: (i, 0)),
        out_shape=jax.ShapeDtypeStruct((N_EDGES, D_EDGE), jnp.float32),
    )(edge_scalar_attr_t, edge_attr_t, W1, W2, Wp)


def _sc_body(feat_hbm, src_hbm, dst_hbm, coeff_hbm,
             out_hbm, bndrow_hbm, bndidx_hbm,
             src4, dst4, coeff4, rows4, stg_rows, stg_idx, bnd_buf, zpeek,
             sem_lin, sem_gath):
    c = lax.axis_index("c")
    s = lax.axis_index("s")
    wid = c * NS + s
    tile_base = wid * EDGES_PER_TILE

    lane = lax.iota(jnp.int32, L)
    zerov = jnp.zeros((L,), jnp.float32)
    lane0 = lane == 0

    def store_idx_scalar(pos, val):
        plsc.store_scatter(stg_idx, [jnp.broadcast_to(pos, (L,))],
                           jnp.broadcast_to(val, (L,)), mask=lane0)

    def lin_start(ci):
        base = tile_base + ci * CHUNK
        s4 = lax.bitwise_and(ci, 3)
        pltpu.async_copy(src_hbm.at[pl.ds(base, CHUNK)], src4.at[s4],
                         sem_lin.at[s4])
        pltpu.async_copy(dst_hbm.at[pl.ds(base, CHUNK)], dst4.at[s4],
                         sem_lin.at[s4])
        pltpu.async_copy(coeff_hbm.at[pl.ds(base, CHUNK)], coeff4.at[s4],
                         sem_lin.at[s4])

    def lin_wait(ci):
        base = tile_base + ci * CHUNK
        s4 = lax.bitwise_and(ci, 3)
        pltpu.make_async_copy(src_hbm.at[pl.ds(base, CHUNK)], src4.at[s4],
                              sem_lin.at[s4]).wait()
        pltpu.make_async_copy(dst_hbm.at[pl.ds(base, CHUNK)], dst4.at[s4],
                              sem_lin.at[s4]).wait()
        pltpu.make_async_copy(coeff_hbm.at[pl.ds(base, CHUNK)], coeff4.at[s4],
                              sem_lin.at[s4]).wait()

    def gather_start(ci):
        s4 = lax.bitwise_and(ci, 3)
        pltpu.async_copy(feat_hbm.at[src4.at[s4]], rows4.at[s4],
                         sem_gath.at[s4])

    def gather_wait(ci):
        s4 = lax.bitwise_and(ci, 3)
        pltpu.make_async_copy(feat_hbm.at[src4.at[s4]], rows4.at[s4],
                              sem_gath.at[s4]).wait()

    lin_start(0)
    lin_start(1)
    lin_start(2)
    lin_wait(0)
    gather_start(0)
    lin_wait(1)
    gather_start(1)

    @pl.when(wid < NW - 1)
    def _():
        pltpu.sync_copy(dst_hbm.at[pl.ds(tile_base + EDGES_PER_TILE, L)],
                        zpeek)

    first_dst = dst4[0, pl.ds(0, L)][0]
    zstart = jnp.where(wid == 0, 0, first_dst)
    zend = jnp.where(wid == NW - 1, N_NODES, zpeek[...][0])

    def zero_stage_row(j, carry):
        for v in range(NV):
            stg_rows[j, pl.ds(v * L, L)] = zerov
        return carry

    lax.fori_loop(0, STG, zero_stage_row, 0, unroll=False)
    for v in range(2 * NV):
        bnd_buf[pl.ds(v * L, L)] = zerov

    def zero_shot(q, carry):
        base_row = zstart + q * STG

        @pl.when(base_row < zend)
        def _():
            for v in range(STG // L):
                vals = jnp.minimum(base_row + v * L + lane, zend - 1)
                stg_idx[pl.ds(v * L, L)] = vals
            pltpu.sync_copy(stg_rows, out_hbm.at[stg_idx])
        return carry

    lax.fori_loop(0, (N_NODES + STG - 1) // STG, zero_shot, 0,
                  unroll=False)

    half = lax.shift_right_logical(lane, 3)
    gdn = lax.GatherDimensionNumbers(offset_dims=(), collapsed_slice_dims=(0,),
                                     start_index_map=(0,))

    def flush_stg(fill, lastidx):
        lastrow = fill - 1

        def pad_row(j, carry):
            @pl.when(j >= fill)
            def _():
                for v in range(NV):
                    stg_rows[j, pl.ds(v * L, L)] = \
                        stg_rows[lastrow, pl.ds(v * L, L)]
            return carry

        lax.fori_loop(0, STG, pad_row, 0, unroll=False)
        lastvec = jnp.broadcast_to(lastidx, (L,))
        for j in range(STG // L):
            curv = stg_idx[pl.ds(j * L, L)]
            stg_idx[pl.ds(j * L, L)] = jnp.where(j * L + lane >= fill,
                                                 lastvec, curv)
        pltpu.sync_copy(stg_rows, out_hbm.at[stg_idx])

    def edge_step(s4, e, d, carry):
        cur, run_idx, fill, bfirst, last_staged, acc = carry
        changed = d != cur

        @pl.when(changed)
        def _():
            @pl.when(run_idx == 0)
            def _():
                for v in range(NV):
                    bnd_buf[pl.ds(v * L, L)] = acc[v]

            @pl.when(run_idx > 0)
            def _():
                for v in range(NV):
                    stg_rows[fill, pl.ds(v * L, L)] = acc[v]
                store_idx_scalar(fill, cur)

        bfirst = jnp.where(changed & (run_idx == 0), cur, bfirst)
        last_staged = jnp.where(changed & (run_idx > 0), cur, last_staged)
        fill = fill + jnp.where(changed & (run_idx > 0), 1, 0)
        run_idx = run_idx + jnp.where(changed, 1, 0)
        cur = jnp.where(changed, d, cur)

        c16 = coeff4[s4, e, :]
        gate = jnp.where(changed, 0.0, 1.0)
        new_acc = []
        for v in range(NV):
            patt = 2 * v + half
            cexp = lax.gather(
                c16, patt[:, None], dimension_numbers=gdn,
                slice_sizes=(1,), indices_are_sorted=True,
                mode=lax.GatherScatterMode.PROMISE_IN_BOUNDS)
            row = rows4[s4, e, pl.ds(v * L, L)]
            new_acc.append(row * cexp + acc[v] * gate)
        return (cur, run_idx, fill, bfirst, last_staged, tuple(new_acc))

    def group_step(s4, g, carry):
        dvec = dst4[s4, pl.ds(g * L, L)]
        for k in range(L):
            carry = edge_step(s4, g * L + k, dvec[k], carry)
        cur, run_idx, fill, bfirst, last_staged, acc = carry

        @pl.when(fill >= CHUNK)
        def _():
            flush_stg(fill, last_staged)

        fill = jnp.where(fill >= CHUNK, 0, fill)
        return (cur, run_idx, fill, bfirst, last_staged, acc)

    def chunk_body(ci, carry):
        s4 = lax.bitwise_and(ci, 3)

        @pl.when(ci + 3 < NCHUNKS)
        def _():
            lin_start(ci + 3)

        @pl.when(ci + 2 < NCHUNKS)
        def _():
            lin_wait(ci + 2)
            gather_start(ci + 2)

        gather_wait(ci)
        return lax.fori_loop(0, CHUNK // L,
                             lambda g, cc: group_step(s4, g, cc),
                             carry, unroll=False)

    acc0 = tuple(zerov for _ in range(NV))
    init = (first_dst, jnp.int32(0), jnp.int32(0), jnp.int32(0), jnp.int32(0),
            acc0)
    cur, run_idx, fill, bfirst, last_staged, acc = lax.fori_loop(
        0, NCHUNKS, chunk_body, init, unroll=False)

    @pl.when(fill > 0)
    def _():
        flush_stg(fill, last_staged)

    for v in range(NV):
        bnd_buf[pl.ds(NV * L + v * L, L)] = acc[v]
    pltpu.sync_copy(bnd_buf, bndrow_hbm.at[pl.ds(wid * 2 * D_FEAT,
                                                 2 * D_FEAT)])

    bvec = jnp.where(lane == 0, bfirst, jnp.where(lane == 1, cur, 0))
    stg_idx[pl.ds(0, L)] = bvec
    pltpu.sync_copy(stg_idx.at[pl.ds(0, L)], bndidx_hbm.at[pl.ds(wid * L, L)])


def _sc_scatter(node_features, edge_src, edge_dst, coeff):
    mesh = plsc.VectorSubcoreMesh(core_axis_name="c", subcore_axis_name="s",
                                  num_cores=NC, num_subcores=NS)
    kern = pl.kernel(
        _sc_body,
        out_type=[
            jax.ShapeDtypeStruct((N_NODES, D_FEAT), jnp.float32),
            jax.ShapeDtypeStruct((NW * 2 * D_FEAT,), jnp.float32),
            jax.ShapeDtypeStruct((NW * L,), jnp.int32),
        ],
        mesh=mesh,
        compiler_params=pltpu.CompilerParams(needs_layout_passes=False),
        scratch_types=[
            pltpu.VMEM((4, CHUNK), jnp.int32),
            pltpu.VMEM((4, CHUNK), jnp.int32),
            pltpu.VMEM((4, CHUNK, D_EDGE), jnp.float32),
            pltpu.VMEM((4, CHUNK, D_FEAT), jnp.float32),
            pltpu.VMEM((STG, D_FEAT), jnp.float32),
            pltpu.VMEM((STG,), jnp.int32),
            pltpu.VMEM((2 * D_FEAT,), jnp.float32),
            pltpu.VMEM((L,), jnp.int32),
            pltpu.SemaphoreType.DMA((4,)),
            pltpu.SemaphoreType.DMA((4,)),
        ],
    )
    return kern(node_features, edge_src, edge_dst, coeff)


def _out_body(agg_ref, bndrow_ref, bndidx_ref, self_ref, wout_ref, out_ref,
              *, blk):
    i = pl.program_id(0)
    c_self = np.float32(np.sqrt(1.0 - MIXING))
    c_conv = np.float32(np.sqrt(MIXING) / np.sqrt(NUM_NEIGHBORS))
    rows = lax.broadcasted_iota(jnp.int32, (blk, 2 * NW), 0) + i * blk
    onehot = (rows == bndidx_ref[...].reshape(1, 2 * NW)).astype(jnp.float32)
    fix = jnp.dot(onehot, bndrow_ref[...], preferred_element_type=jnp.float32)
    agg = agg_ref[...] + fix
    conv = jnp.dot(agg, wout_ref[...], preferred_element_type=jnp.float32)
    out_ref[...] = c_self * self_ref[...] + c_conv * conv


def _out_linear(agg, bnd_rows, bnd_idx, node_self_out, W_out):
    blk = 1000
    grid = N_NODES // blk
    return pl.pallas_call(
        functools.partial(_out_body, blk=blk),
        grid=(grid,),
        in_specs=[
            pl.BlockSpec((blk, D_FEAT), lambda i: (i, 0)),
            pl.BlockSpec((2 * NW, D_FEAT), lambda i: (0, 0)),
            pl.BlockSpec((1, 2 * NW), lambda i: (0, 0)),
            pl.BlockSpec((blk, D_OUT), lambda i: (i, 0)),
            pl.BlockSpec((D_FEAT, D_OUT), lambda i: (0, 0)),
        ],
        out_specs=pl.BlockSpec((blk, D_OUT), lambda i: (i, 0)),
        out_shape=jax.ShapeDtypeStruct((N_NODES, D_OUT), jnp.float32),
    )(agg, bnd_rows, bnd_idx, node_self_out, W_out)


@jax.jit
def kernel(node_input, edge_src, edge_dst, edge_attr, edge_scalar_attr,
           W_node, W1, W2, Wp, W_out):
    node_features, node_self_out = _node_linear(node_input, W_node)
    coeff = _radial(edge_scalar_attr.T, edge_attr.T, W1, W2, Wp)
    agg, bnd_pair, bnd_idx = _sc_scatter(node_features, edge_src, edge_dst,
                                         coeff)
    bnd_rows = bnd_pair.reshape(NW * 2, D_FEAT)
    bidx = bnd_idx.reshape(NW, L)[:, :2].reshape(1, 2 * NW)
    return _out_linear(agg, bnd_rows, bidx, node_self_out, W_out)
